# Initial kernel scaffold; baseline (speedup 1.0000x reference)
import jax
import jax.numpy as jnp
from jax.experimental import pallas as pl


def kernel(x, edge_index, batch, mask, W1, b1, W2, b2, Wl1, bl1, Wl2, bl2):
    raise NotImplementedError("write your pallas kernel here")


# speedup vs baseline: 12.2635x; 12.2635x over previous
import functools

import jax
import jax.numpy as jnp
import numpy as np
from jax import lax
from jax.experimental import pallas as pl
from jax.experimental.pallas import tpu as pltpu
from jax.experimental.pallas import tpu_sc as plsc

N, E, D, G = 10000, 320000, 128, 64
NC, NS = 2, 16
NW = NC * NS
EPT = E // NW
CHUNK = 80
NCHUNK = EPT // CHUNK
NP = 10240
RPT = NP // NS
BN = 400
NB = N // BN
NEG = np.float32(-1000000000.0)

def _mesh():
    return plsc.VectorSubcoreMesh(core_axis_name="c", subcore_axis_name="s")


def _deg_sc(dst3, zeros16):
    @functools.partial(
        pl.kernel,
        out_type=jax.ShapeDtypeStruct((NC, NP, 16), jnp.float32),
        mesh=_mesh(),
        scratch_types=[
            pltpu.VMEM((NCHUNK, CHUNK), jnp.int32),
            pltpu.VMEM((CHUNK, 16), jnp.float32),
            pltpu.VMEM_SHARED((NP, 16), jnp.float32),
        ],
    )
    def k(dst_hbm, z_hbm, out_hbm, idx_v, ones_v, acc_sh):
        c = lax.axis_index("c")
        s = lax.axis_index("s")
        tile = c * NS + s

        @pl.loop(0, CHUNK)
        def _(i):
            ones_v[i, :] = jnp.full((16,), 1.0, jnp.float32)

        @pl.when(s == 0)
        def _():
            pltpu.sync_copy(z_hbm, acc_sh)

        pltpu.sync_copy(dst_hbm.at[tile], idx_v)
        plsc.subcore_barrier()

        @pl.loop(0, NCHUNK)
        def _(j):
            pltpu.sync_copy(ones_v, acc_sh.at[idx_v.at[j]], add=True)

        plsc.subcore_barrier()

        @pl.when(s == 0)
        def _():
            pltpu.sync_copy(acc_sh, out_hbm.at[c])

    return k(dst3, zeros16)


def _edge_sc(hp, src3, dst3, zerosD):
    @functools.partial(
        pl.kernel,
        out_type=jax.ShapeDtypeStruct((NC, NP, D), jnp.float32),
        mesh=_mesh(),
        scratch_types=[
            pltpu.VMEM((CHUNK,), jnp.int32),
            pltpu.VMEM((CHUNK,), jnp.int32),
            pltpu.VMEM((CHUNK, D), jnp.float32),
            pltpu.VMEM_SHARED((NP, D), jnp.float32),
        ],
    )
    def k(hp_hbm, src_hbm, dst_hbm, z_hbm, out_hbm, ibs_v, ibd_v, rows_v,
          acc_sh):
        c = lax.axis_index("c")
        s = lax.axis_index("s")
        tile = c * NS + s

        @pl.when(s == 0)
        def _():
            pltpu.sync_copy(z_hbm, acc_sh)

        plsc.subcore_barrier()

        @pl.loop(0, NCHUNK)
        def _(j):
            pltpu.sync_copy(src_hbm.at[tile, j], ibs_v)
            pltpu.sync_copy(dst_hbm.at[tile, j], ibd_v)
            pltpu.sync_copy(hp_hbm.at[ibs_v], rows_v)
            pltpu.sync_copy(rows_v, acc_sh.at[ibd_v], add=True)

        plsc.subcore_barrier()

        @pl.when(s == 0)
        def _():
            pltpu.sync_copy(acc_sh, out_hbm.at[c])

    return k(hp, src3, dst3, zerosD)


def _l1_tc(x, W1, degp):
    def body(x_ref, w_ref, deg_ref, out_ref):
        d = jnp.sum(deg_ref[...], axis=(0, 2)) + 1.0
        dis = lax.rsqrt(d)
        h = jnp.dot(x_ref[...], w_ref[...],
                    preferred_element_type=jnp.float32,
                    precision=lax.Precision.HIGHEST)
        out_ref[...] = h * dis[:, None]

    return pl.pallas_call(
        body,
        grid=(NB,),
        in_specs=[
            pl.BlockSpec((BN, D), lambda i: (i, 0)),
            pl.BlockSpec((D, D), lambda i: (0, 0)),
            pl.BlockSpec((NC, BN, 16), lambda i: (0, i, 0)),
        ],
        out_specs=pl.BlockSpec((BN, D), lambda i: (i, 0)),
        out_shape=jax.ShapeDtypeStruct((N, D), jnp.float32),
    )(x, W1, degp)


def _l2_tc(S1, h1p, degp, b1r, W2):
    def body(s1_ref, h1p_ref, deg_ref, b1_ref, w_ref, out_ref):
        d = jnp.sum(deg_ref[...], axis=(0, 2)) + 1.0
        dis = lax.rsqrt(d)
        tot = s1_ref[0] + s1_ref[1] + h1p_ref[...]
        h1 = jnp.maximum(tot * dis[:, None] + b1_ref[...], 0.0)
        h2 = jnp.dot(h1, w_ref[...],
                     preferred_element_type=jnp.float32,
                     precision=lax.Precision.HIGHEST)
        out_ref[...] = h2 * dis[:, None]

    return pl.pallas_call(
        body,
        grid=(NB,),
        in_specs=[
            pl.BlockSpec((NC, BN, D), lambda i: (0, i, 0)),
            pl.BlockSpec((BN, D), lambda i: (i, 0)),
            pl.BlockSpec((NC, BN, 16), lambda i: (0, i, 0)),
            pl.BlockSpec((1, D), lambda i: (0, 0)),
            pl.BlockSpec((D, D), lambda i: (0, 0)),
        ],
        out_specs=pl.BlockSpec((BN, D), lambda i: (i, 0)),
        out_shape=jax.ShapeDtypeStruct((N, D), jnp.float32),
    )(S1, h1p, degp, b1r, W2)


def _score_tc(S2, h2p, degp, b2r, Wl1, bl1r, Wl2, bl2r, mask3):
    def body(s2_ref, h2p_ref, deg_ref, b2_ref, wl1_ref, bl1_ref, wl2_ref,
             bl2_ref, m_ref, out_ref):
        d = jnp.sum(deg_ref[...], axis=(0, 2)) + 1.0
        dis = lax.rsqrt(d)
        tot = s2_ref[0] + s2_ref[1] + h2p_ref[...]
        h2 = tot * dis[:, None] + b2_ref[...]
        t = jnp.maximum(
            jnp.dot(h2, wl1_ref[...],
                    preferred_element_type=jnp.float32,
                    precision=lax.Precision.HIGHEST) + bl1_ref[...], 0.0)
        sv = jnp.dot(t, wl2_ref[...],
                     preferred_element_type=jnp.float32,
                     precision=lax.Precision.HIGHEST)
        sc = sv[:, 0] + bl2_ref[0, 0]
        sc = jnp.where(m_ref[0, 0, :] == 0, NEG, sc)
        out_ref[0, 0, :] = sc

    return pl.pallas_call(
        body,
        grid=(NB,),
        in_specs=[
            pl.BlockSpec((NC, BN, D), lambda i: (0, i, 0)),
            pl.BlockSpec((BN, D), lambda i: (i, 0)),
            pl.BlockSpec((NC, BN, 16), lambda i: (0, i, 0)),
            pl.BlockSpec((1, D), lambda i: (0, 0)),
            pl.BlockSpec((D, D), lambda i: (0, 0)),
            pl.BlockSpec((1, D), lambda i: (0, 0)),
            pl.BlockSpec((D, 1), lambda i: (0, 0)),
            pl.BlockSpec((1, 1), lambda i: (0, 0)),
            pl.BlockSpec((1, 1, BN), lambda i: (i, 0, 0)),
        ],
        out_specs=pl.BlockSpec((1, 1, BN), lambda i: (i, 0, 0)),
        out_shape=jax.ShapeDtypeStruct((NB, 1, BN), jnp.float32),
    )(S2, h2p, degp, b2r, Wl1, bl1r, Wl2, bl2r, mask3)


def _softmax_tc(sarr, batch3):
    def body(s_ref, b_ref, out_ref, m_sc, den_sc, ex_sc):
        p = pl.program_id(0)
        j = pl.program_id(1)
        sblk = s_ref[0, 0, :]
        bblk = b_ref[0, 0, :]
        eq = bblk[:, None] == lax.broadcasted_iota(jnp.int32, (BN, G), 1)

        @pl.when(jnp.logical_and(p == 0, j == 0))
        def _():
            m_sc[...] = jnp.full((1, G), -3.0e38, jnp.float32)

        @pl.when(p == 0)
        def _():
            contrib = jnp.max(jnp.where(eq, sblk[:, None], -3.0e38), axis=0)
            m_sc[...] = jnp.maximum(m_sc[...], contrib[None, :])

        @pl.when(jnp.logical_and(p == 1, j == 0))
        def _():
            den_sc[...] = jnp.zeros((1, G), jnp.float32)

        @pl.when(p == 1)
        def _():
            mb = jnp.broadcast_to(m_sc[...], (BN, G))
            mpn = jnp.sum(jnp.where(eq, mb, 0.0), axis=1)
            ex = jnp.exp(sblk - mpn)
            ex_sc[j, 0, :] = ex
            den_sc[...] = den_sc[...] + jnp.sum(
                jnp.where(eq, ex[:, None], 0.0), axis=0)[None, :]

        @pl.when(p == 2)
        def _():
            db = jnp.broadcast_to(den_sc[...], (BN, G))
            dpn = jnp.sum(jnp.where(eq, db, 0.0), axis=1)
            out_ref[0, 0, :] = ex_sc[j, 0, :] / dpn

    return pl.pallas_call(
        body,
        grid=(3, NB),
        in_specs=[
            pl.BlockSpec((1, 1, BN), lambda p, j: (j, 0, 0)),
            pl.BlockSpec((1, 1, BN), lambda p, j: (j, 0, 0)),
        ],
        out_specs=pl.BlockSpec((1, 1, BN), lambda p, j: (j, 0, 0)),
        out_shape=jax.ShapeDtypeStruct((NB, 1, BN), jnp.float32),
        scratch_shapes=[
            pltpu.VMEM((1, G), jnp.float32),
            pltpu.VMEM((1, G), jnp.float32),
            pltpu.VMEM((NB, 1, BN), jnp.float32),
        ],
    )(sarr, batch3)


def kernel(x, edge_index, batch, mask, W1, b1, W2, b2, Wl1, bl1, Wl2, bl2):
    src3 = edge_index[0].reshape(NW, NCHUNK, CHUNK)
    dst3 = edge_index[1].reshape(NW, NCHUNK, CHUNK)
    zeros16 = jnp.zeros((NP, 16), jnp.float32)
    zerosD = jnp.zeros((NP, D), jnp.float32)

    degp = _deg_sc(dst3, zeros16)
    h1p = _l1_tc(x, W1, degp)
    S1 = _edge_sc(h1p, src3, dst3, zerosD)
    h2p = _l2_tc(S1, h1p, degp, b1.reshape(1, D), W2)
    S2 = _edge_sc(h2p, src3, dst3, zerosD)
    sarr = _score_tc(S2, h2p, degp, b2.reshape(1, D), Wl1,
                     bl1.reshape(1, D), Wl2, bl2.reshape(1, 1),
                     mask.reshape(NB, 1, BN))
    out3 = _softmax_tc(sarr, batch.reshape(NB, 1, BN))
    return out3.reshape(N)


# speedup vs baseline: 14.1979x; 1.1577x over previous
import functools

import jax
import jax.numpy as jnp
import numpy as np
from jax import lax
from jax.experimental import pallas as pl
from jax.experimental.pallas import tpu as pltpu
from jax.experimental.pallas import tpu_sc as plsc

N, E, D, G = 10000, 320000, 128, 64
NC, NS = 2, 16
NW = NC * NS
EPT = E // NW
CHUNK = 80
NCHUNK = EPT // CHUNK
NP = 10240
RPT = NP // NS
BN = 400
NB = N // BN
NEG = np.float32(-1000000000.0)

def _mesh():
    return plsc.VectorSubcoreMesh(core_axis_name="c", subcore_axis_name="s")


def _deg_sc(dst3, zeros16):
    @functools.partial(
        pl.kernel,
        out_type=jax.ShapeDtypeStruct((NC, NP, 16), jnp.float32),
        mesh=_mesh(),
        scratch_types=[
            pltpu.VMEM((NCHUNK, CHUNK), jnp.int32),
            pltpu.VMEM((CHUNK, 16), jnp.float32),
            pltpu.VMEM_SHARED((NP, 16), jnp.float32),
        ],
    )
    def k(dst_hbm, z_hbm, out_hbm, idx_v, ones_v, acc_sh):
        c = lax.axis_index("c")
        s = lax.axis_index("s")
        tile = c * NS + s

        @pl.loop(0, CHUNK)
        def _(i):
            ones_v[i, :] = jnp.full((16,), 1.0, jnp.float32)

        @pl.when(s == 0)
        def _():
            pltpu.sync_copy(z_hbm, acc_sh)

        pltpu.sync_copy(dst_hbm.at[tile], idx_v)
        plsc.subcore_barrier()

        @pl.loop(0, NCHUNK)
        def _(j):
            pltpu.sync_copy(ones_v, acc_sh.at[idx_v.at[j]], add=True)

        plsc.subcore_barrier()

        @pl.when(s == 0)
        def _():
            pltpu.sync_copy(acc_sh, out_hbm.at[c])

    return k(dst3, zeros16)


def _edge_sc(hp, src3, dst3, zerosD):
    @functools.partial(
        pl.kernel,
        out_type=jax.ShapeDtypeStruct((NC, NP, D), jnp.float32),
        mesh=_mesh(),
        scratch_types=[
            pltpu.VMEM((CHUNK,), jnp.int32),
            pltpu.VMEM((CHUNK,), jnp.int32),
            pltpu.VMEM((CHUNK,), jnp.int32),
            pltpu.VMEM((CHUNK,), jnp.int32),
            pltpu.VMEM((CHUNK, D), jnp.float32),
            pltpu.VMEM((CHUNK, D), jnp.float32),
            pltpu.VMEM_SHARED((NP, D), jnp.float32),
            pltpu.SemaphoreType.DMA,
            pltpu.SemaphoreType.DMA,
            pltpu.SemaphoreType.DMA,
            pltpu.SemaphoreType.DMA,
            pltpu.SemaphoreType.DMA,
            pltpu.SemaphoreType.DMA,
        ],
    )
    def k(hp_hbm, src_hbm, dst_hbm, z_hbm, out_hbm,
          ibs0, ibs1, ibd0, ibd1, rows0, rows1, acc_sh,
          is0, is1, id0, id1, g0, g1):
        c = lax.axis_index("c")
        s = lax.axis_index("s")
        tile = c * NS + s
        tbase = tile * NCHUNK
        ibs = (ibs0, ibs1)
        ibd = (ibd0, ibd1)
        rows = (rows0, rows1)
        isem = (is0, is1)
        dsem = (id0, id1)
        gsem = (g0, g1)

        @pl.when(s == 0)
        def _():
            pltpu.sync_copy(z_hbm, acc_sh)

        plsc.subcore_barrier()

        def idx_start(j, b):
            pltpu.async_copy(src_hbm.at[tbase + j], ibs[b], isem[b])
            pltpu.async_copy(dst_hbm.at[tbase + j], ibd[b], dsem[b])

        def idx_wait_s(j, b):
            pltpu.make_async_copy(src_hbm.at[tbase + j], ibs[b], isem[b]).wait()

        def idx_wait_d(j, b):
            pltpu.make_async_copy(dst_hbm.at[tbase + j], ibd[b], dsem[b]).wait()

        def gather_start(b):
            del b

        def gather_wait(b):
            pltpu.sync_copy(hp_hbm.at[ibs[b]], rows[b])

        idx_start(0, 0)
        idx_start(1, 1)
        idx_wait_s(0, 0)
        gather_start(0)

        @pl.loop(0, NCHUNK // 2)
        def _(g):
            j = 2 * g
            idx_wait_s(j + 1, 1)
            gather_start(1)
            gather_wait(0)
            idx_wait_d(j, 0)
            pltpu.sync_copy(rows0, acc_sh.at[ibd0], add=True)
            idx_start(j + 2, 0)
            @pl.when(j + 2 < NCHUNK)
            def _():
                idx_wait_s(j + 2, 0)
                gather_start(0)
            gather_wait(1)
            idx_wait_d(j + 1, 1)
            pltpu.sync_copy(rows1, acc_sh.at[ibd1], add=True)

            @pl.when(j + 3 < NCHUNK)
            def _():
                idx_start(j + 3, 1)

        gather_wait(0)
        idx_wait_d(NCHUNK - 1, 0)
        pltpu.sync_copy(rows0, acc_sh.at[ibd0], add=True)

        plsc.subcore_barrier()

        @pl.when(s == 0)
        def _():
            pltpu.sync_copy(acc_sh, out_hbm.at[c])

    return k(hp, src3.reshape(NW * NCHUNK, CHUNK),
             dst3.reshape(NW * NCHUNK, CHUNK), zerosD)


def _l1_tc(x, W1, degp):
    def body(x_ref, w_ref, deg_ref, out_ref):
        d = jnp.sum(deg_ref[...], axis=(0, 2)) + 1.0
        dis = lax.rsqrt(d)
        h = jnp.dot(x_ref[...], w_ref[...],
                    preferred_element_type=jnp.float32,
                    precision=lax.Precision.HIGHEST)
        out_ref[...] = h * dis[:, None]

    return pl.pallas_call(
        body,
        grid=(NB,),
        in_specs=[
            pl.BlockSpec((BN, D), lambda i: (i, 0)),
            pl.BlockSpec((D, D), lambda i: (0, 0)),
            pl.BlockSpec((NC, BN, 16), lambda i: (0, i, 0)),
        ],
        out_specs=pl.BlockSpec((BN, D), lambda i: (i, 0)),
        out_shape=jax.ShapeDtypeStruct((N, D), jnp.float32),
    )(x, W1, degp)


def _l2_tc(S1, h1p, degp, b1r, W2):
    def body(s1_ref, h1p_ref, deg_ref, b1_ref, w_ref, out_ref):
        d = jnp.sum(deg_ref[...], axis=(0, 2)) + 1.0
        dis = lax.rsqrt(d)
        tot = s1_ref[0] + s1_ref[1] + h1p_ref[...]
        h1 = jnp.maximum(tot * dis[:, None] + b1_ref[...], 0.0)
        h2 = jnp.dot(h1, w_ref[...],
                     preferred_element_type=jnp.float32,
                     precision=lax.Precision.HIGHEST)
        out_ref[...] = h2 * dis[:, None]

    return pl.pallas_call(
        body,
        grid=(NB,),
        in_specs=[
            pl.BlockSpec((NC, BN, D), lambda i: (0, i, 0)),
            pl.BlockSpec((BN, D), lambda i: (i, 0)),
            pl.BlockSpec((NC, BN, 16), lambda i: (0, i, 0)),
            pl.BlockSpec((1, D), lambda i: (0, 0)),
            pl.BlockSpec((D, D), lambda i: (0, 0)),
        ],
        out_specs=pl.BlockSpec((BN, D), lambda i: (i, 0)),
        out_shape=jax.ShapeDtypeStruct((N, D), jnp.float32),
    )(S1, h1p, degp, b1r, W2)


def _score_tc(S2, h2p, degp, b2r, Wl1, bl1r, Wl2, bl2r, mask3):
    def body(s2_ref, h2p_ref, deg_ref, b2_ref, wl1_ref, bl1_ref, wl2_ref,
             bl2_ref, m_ref, out_ref):
        d = jnp.sum(deg_ref[...], axis=(0, 2)) + 1.0
        dis = lax.rsqrt(d)
        tot = s2_ref[0] + s2_ref[1] + h2p_ref[...]
        h2 = tot * dis[:, None] + b2_ref[...]
        t = jnp.maximum(
            jnp.dot(h2, wl1_ref[...],
                    preferred_element_type=jnp.float32,
                    precision=lax.Precision.HIGHEST) + bl1_ref[...], 0.0)
        sv = jnp.dot(t, wl2_ref[...],
                     preferred_element_type=jnp.float32,
                     precision=lax.Precision.HIGHEST)
        sc = sv[:, 0] + bl2_ref[0, 0]
        sc = jnp.where(m_ref[0, 0, :] == 0, NEG, sc)
        out_ref[0, 0, :] = sc

    return pl.pallas_call(
        body,
        grid=(NB,),
        in_specs=[
            pl.BlockSpec((NC, BN, D), lambda i: (0, i, 0)),
            pl.BlockSpec((BN, D), lambda i: (i, 0)),
            pl.BlockSpec((NC, BN, 16), lambda i: (0, i, 0)),
            pl.BlockSpec((1, D), lambda i: (0, 0)),
            pl.BlockSpec((D, D), lambda i: (0, 0)),
            pl.BlockSpec((1, D), lambda i: (0, 0)),
            pl.BlockSpec((D, 1), lambda i: (0, 0)),
            pl.BlockSpec((1, 1), lambda i: (0, 0)),
            pl.BlockSpec((1, 1, BN), lambda i: (i, 0, 0)),
        ],
        out_specs=pl.BlockSpec((1, 1, BN), lambda i: (i, 0, 0)),
        out_shape=jax.ShapeDtypeStruct((NB, 1, BN), jnp.float32),
    )(S2, h2p, degp, b2r, Wl1, bl1r, Wl2, bl2r, mask3)


def _softmax_tc(sarr, batch3):
    def body(s_ref, b_ref, out_ref, m_sc, den_sc, ex_sc):
        p = pl.program_id(0)
        j = pl.program_id(1)
        sblk = s_ref[0, 0, :]
        bblk = b_ref[0, 0, :]
        eq = bblk[:, None] == lax.broadcasted_iota(jnp.int32, (BN, G), 1)

        @pl.when(jnp.logical_and(p == 0, j == 0))
        def _():
            m_sc[...] = jnp.full((1, G), -3.0e38, jnp.float32)

        @pl.when(p == 0)
        def _():
            contrib = jnp.max(jnp.where(eq, sblk[:, None], -3.0e38), axis=0)
            m_sc[...] = jnp.maximum(m_sc[...], contrib[None, :])

        @pl.when(jnp.logical_and(p == 1, j == 0))
        def _():
            den_sc[...] = jnp.zeros((1, G), jnp.float32)

        @pl.when(p == 1)
        def _():
            mb = jnp.broadcast_to(m_sc[...], (BN, G))
            mpn = jnp.sum(jnp.where(eq, mb, 0.0), axis=1)
            ex = jnp.exp(sblk - mpn)
            ex_sc[j, 0, :] = ex
            den_sc[...] = den_sc[...] + jnp.sum(
                jnp.where(eq, ex[:, None], 0.0), axis=0)[None, :]

        @pl.when(p == 2)
        def _():
            db = jnp.broadcast_to(den_sc[...], (BN, G))
            dpn = jnp.sum(jnp.where(eq, db, 0.0), axis=1)
            out_ref[0, 0, :] = ex_sc[j, 0, :] / dpn

    return pl.pallas_call(
        body,
        grid=(3, NB),
        in_specs=[
            pl.BlockSpec((1, 1, BN), lambda p, j: (j, 0, 0)),
            pl.BlockSpec((1, 1, BN), lambda p, j: (j, 0, 0)),
        ],
        out_specs=pl.BlockSpec((1, 1, BN), lambda p, j: (j, 0, 0)),
        out_shape=jax.ShapeDtypeStruct((NB, 1, BN), jnp.float32),
        scratch_shapes=[
            pltpu.VMEM((1, G), jnp.float32),
            pltpu.VMEM((1, G), jnp.float32),
            pltpu.VMEM((NB, 1, BN), jnp.float32),
        ],
    )(sarr, batch3)


def kernel(x, edge_index, batch, mask, W1, b1, W2, b2, Wl1, bl1, Wl2, bl2):
    src3 = edge_index[0].reshape(NW, NCHUNK, CHUNK)
    dst3 = edge_index[1].reshape(NW, NCHUNK, CHUNK)
    zeros16 = jnp.zeros((NP, 16), jnp.float32)
    zerosD = jnp.zeros((NP, D), jnp.float32)

    degp = _deg_sc(dst3, zeros16)
    h1p = _l1_tc(x, W1, degp)
    S1 = _edge_sc(h1p, src3, dst3, zerosD)
    h2p = _l2_tc(S1, h1p, degp, b1.reshape(1, D), W2)
    S2 = _edge_sc(h2p, src3, dst3, zerosD)
    sarr = _score_tc(S2, h2p, degp, b2.reshape(1, D), Wl1,
                     bl1.reshape(1, D), Wl2, bl2.reshape(1, 1),
                     mask.reshape(NB, 1, BN))
    out3 = _softmax_tc(sarr, batch.reshape(NB, 1, BN))
    return out3.reshape(N)


# speedup vs baseline: 20.2463x; 1.4260x over previous
import functools

import jax
import jax.numpy as jnp
import numpy as np
from jax import lax
from jax.experimental import pallas as pl
from jax.experimental.pallas import tpu as pltpu
from jax.experimental.pallas import tpu_sc as plsc

N, E, D, G = 10000, 320000, 128, 64
NC, NS = 2, 16
NW = NC * NS
EPT = E // NW
CHUNK = 80
NCHUNK = EPT // CHUNK
NP = 10240
RPT = NP // NS
BN = 400
NB = N // BN
NEG = np.float32(-1000000000.0)

def _mesh():
    return plsc.VectorSubcoreMesh(core_axis_name="c", subcore_axis_name="s")


def _deg_sc(dst3, zeros16):
    @functools.partial(
        pl.kernel,
        out_type=jax.ShapeDtypeStruct((NC, NP, 16), jnp.float32),
        mesh=_mesh(),
        scratch_types=[
            pltpu.VMEM((NCHUNK, CHUNK), jnp.int32),
            pltpu.VMEM((CHUNK, 16), jnp.float32),
            pltpu.VMEM_SHARED((NP, 16), jnp.float32),
        ],
    )
    def k(dst_hbm, z_hbm, out_hbm, idx_v, ones_v, acc_sh):
        c = lax.axis_index("c")
        s = lax.axis_index("s")
        tile = c * NS + s

        @pl.loop(0, CHUNK)
        def _(i):
            ones_v[i, :] = jnp.full((16,), 1.0, jnp.float32)

        @pl.when(s == 0)
        def _():
            pltpu.sync_copy(z_hbm, acc_sh)

        pltpu.sync_copy(dst_hbm.at[tile], idx_v)
        plsc.subcore_barrier()

        @pl.loop(0, NCHUNK)
        def _(j):
            pltpu.sync_copy(ones_v, acc_sh.at[idx_v.at[j]], add=True)

        plsc.subcore_barrier()

        @pl.when(s == 0)
        def _():
            pltpu.sync_copy(acc_sh, out_hbm.at[c])

    return k(dst3, zeros16)


def _edge_sc(hp, src3, dst3, zerosD):
    @functools.partial(
        pl.kernel,
        out_type=jax.ShapeDtypeStruct((NC, NP, D), jnp.float32),
        mesh=_mesh(),
        scratch_types=[
            pltpu.VMEM((CHUNK,), jnp.int32),
            pltpu.VMEM((CHUNK,), jnp.int32),
            pltpu.VMEM((CHUNK,), jnp.int32),
            pltpu.VMEM((CHUNK,), jnp.int32),
            pltpu.VMEM((CHUNK, D), jnp.float32),
            pltpu.VMEM((CHUNK, D), jnp.float32),
            pltpu.VMEM_SHARED((NP, D), jnp.float32),
            pltpu.SemaphoreType.DMA,
            pltpu.SemaphoreType.DMA,
            pltpu.SemaphoreType.DMA,
            pltpu.SemaphoreType.DMA,
            pltpu.SemaphoreType.DMA,
            pltpu.SemaphoreType.DMA,
        ],
    )
    def k(hp_hbm, src_hbm, dst_hbm, z_hbm, out_hbm,
          ibs0, ibs1, ibd0, ibd1, rows0, rows1, acc_sh,
          is0, is1, id0, id1, g0, g1):
        c = lax.axis_index("c")
        s = lax.axis_index("s")
        tile = c * NS + s
        tbase = tile * NCHUNK
        ibs = (ibs0, ibs1)
        ibd = (ibd0, ibd1)
        rows = (rows0, rows1)
        isem = (is0, is1)
        dsem = (id0, id1)
        gsem = (g0, g1)

        @pl.when(s == 0)
        def _():
            pltpu.sync_copy(z_hbm, acc_sh)

        plsc.subcore_barrier()

        def idx_start(j, b):
            pltpu.async_copy(src_hbm.at[tbase + j], ibs[b], isem[b])
            pltpu.async_copy(dst_hbm.at[tbase + j], ibd[b], dsem[b])

        def idx_wait_s(j, b):
            pltpu.make_async_copy(src_hbm.at[tbase + j], ibs[b], isem[b]).wait()

        def idx_wait_d(j, b):
            pltpu.make_async_copy(dst_hbm.at[tbase + j], ibd[b], dsem[b]).wait()

        def gather_start(b):
            pltpu.async_copy(hp_hbm.at[ibs[b]], rows[b], gsem[b])

        def gather_wait(b):
            pltpu.make_async_copy(hp_hbm.at[ibs[b]], rows[b], gsem[b]).wait()

        idx_start(0, 0)
        idx_start(1, 1)
        idx_wait_s(0, 0)
        gather_start(0)

        @pl.loop(0, NCHUNK // 2)
        def _(g):
            j = 2 * g
            gather_wait(0)
            idx_wait_s(j + 1, 1)
            gather_start(1)
            idx_wait_d(j, 0)
            pltpu.sync_copy(rows0, acc_sh.at[ibd0], add=True)
            idx_start(j + 2, 0)
            gather_wait(1)

            @pl.when(j + 2 < NCHUNK)
            def _():
                idx_wait_s(j + 2, 0)
                gather_start(0)
            idx_wait_d(j + 1, 1)
            pltpu.sync_copy(rows1, acc_sh.at[ibd1], add=True)

            @pl.when(j + 3 < NCHUNK)
            def _():
                idx_start(j + 3, 1)

        gather_wait(0)
        idx_wait_d(NCHUNK - 1, 0)
        pltpu.sync_copy(rows0, acc_sh.at[ibd0], add=True)

        plsc.subcore_barrier()

        @pl.when(s == 0)
        def _():
            pltpu.sync_copy(acc_sh, out_hbm.at[c])

    return k(hp, src3.reshape(NW * NCHUNK, CHUNK),
             dst3.reshape(NW * NCHUNK, CHUNK), zerosD)


def _l1_tc(x, W1, degp):
    def body(x_ref, w_ref, deg_ref, out_ref):
        d = jnp.sum(deg_ref[...], axis=(0, 2)) + 1.0
        dis = lax.rsqrt(d)
        h = jnp.dot(x_ref[...], w_ref[...],
                    preferred_element_type=jnp.float32,
                    precision=lax.Precision.HIGHEST)
        out_ref[...] = h * dis[:, None]

    return pl.pallas_call(
        body,
        grid=(NB,),
        in_specs=[
            pl.BlockSpec((BN, D), lambda i: (i, 0)),
            pl.BlockSpec((D, D), lambda i: (0, 0)),
            pl.BlockSpec((NC, BN, 16), lambda i: (0, i, 0)),
        ],
        out_specs=pl.BlockSpec((BN, D), lambda i: (i, 0)),
        out_shape=jax.ShapeDtypeStruct((N, D), jnp.float32),
    )(x, W1, degp)


def _l2_tc(S1, h1p, degp, b1r, W2):
    def body(s1_ref, h1p_ref, deg_ref, b1_ref, w_ref, out_ref):
        d = jnp.sum(deg_ref[...], axis=(0, 2)) + 1.0
        dis = lax.rsqrt(d)
        tot = s1_ref[0] + s1_ref[1] + h1p_ref[...]
        h1 = jnp.maximum(tot * dis[:, None] + b1_ref[...], 0.0)
        h2 = jnp.dot(h1, w_ref[...],
                     preferred_element_type=jnp.float32,
                     precision=lax.Precision.HIGHEST)
        out_ref[...] = h2 * dis[:, None]

    return pl.pallas_call(
        body,
        grid=(NB,),
        in_specs=[
            pl.BlockSpec((NC, BN, D), lambda i: (0, i, 0)),
            pl.BlockSpec((BN, D), lambda i: (i, 0)),
            pl.BlockSpec((NC, BN, 16), lambda i: (0, i, 0)),
            pl.BlockSpec((1, D), lambda i: (0, 0)),
            pl.BlockSpec((D, D), lambda i: (0, 0)),
        ],
        out_specs=pl.BlockSpec((BN, D), lambda i: (i, 0)),
        out_shape=jax.ShapeDtypeStruct((N, D), jnp.float32),
    )(S1, h1p, degp, b1r, W2)


def _score_tc(S2, h2p, degp, b2r, Wl1, bl1r, Wl2, bl2r, mask3):
    def body(s2_ref, h2p_ref, deg_ref, b2_ref, wl1_ref, bl1_ref, wl2_ref,
             bl2_ref, m_ref, out_ref):
        d = jnp.sum(deg_ref[...], axis=(0, 2)) + 1.0
        dis = lax.rsqrt(d)
        tot = s2_ref[0] + s2_ref[1] + h2p_ref[...]
        h2 = tot * dis[:, None] + b2_ref[...]
        t = jnp.maximum(
            jnp.dot(h2, wl1_ref[...],
                    preferred_element_type=jnp.float32,
                    precision=lax.Precision.HIGHEST) + bl1_ref[...], 0.0)
        sv = jnp.dot(t, wl2_ref[...],
                     preferred_element_type=jnp.float32,
                     precision=lax.Precision.HIGHEST)
        sc = sv[:, 0] + bl2_ref[0, 0]
        sc = jnp.where(m_ref[0, 0, :] == 0, NEG, sc)
        out_ref[0, 0, :] = sc

    return pl.pallas_call(
        body,
        grid=(NB,),
        in_specs=[
            pl.BlockSpec((NC, BN, D), lambda i: (0, i, 0)),
            pl.BlockSpec((BN, D), lambda i: (i, 0)),
            pl.BlockSpec((NC, BN, 16), lambda i: (0, i, 0)),
            pl.BlockSpec((1, D), lambda i: (0, 0)),
            pl.BlockSpec((D, D), lambda i: (0, 0)),
            pl.BlockSpec((1, D), lambda i: (0, 0)),
            pl.BlockSpec((D, 1), lambda i: (0, 0)),
            pl.BlockSpec((1, 1), lambda i: (0, 0)),
            pl.BlockSpec((1, 1, BN), lambda i: (i, 0, 0)),
        ],
        out_specs=pl.BlockSpec((1, 1, BN), lambda i: (i, 0, 0)),
        out_shape=jax.ShapeDtypeStruct((NB, 1, BN), jnp.float32),
    )(S2, h2p, degp, b2r, Wl1, bl1r, Wl2, bl2r, mask3)


def _softmax_tc(sarr, batch3):
    def body(s_ref, b_ref, out_ref, m_sc, den_sc, ex_sc):
        p = pl.program_id(0)
        j = pl.program_id(1)
        sblk = s_ref[0, 0, :]
        bblk = b_ref[0, 0, :]
        eq = bblk[:, None] == lax.broadcasted_iota(jnp.int32, (BN, G), 1)

        @pl.when(jnp.logical_and(p == 0, j == 0))
        def _():
            m_sc[...] = jnp.full((1, G), -3.0e38, jnp.float32)

        @pl.when(p == 0)
        def _():
            contrib = jnp.max(jnp.where(eq, sblk[:, None], -3.0e38), axis=0)
            m_sc[...] = jnp.maximum(m_sc[...], contrib[None, :])

        @pl.when(jnp.logical_and(p == 1, j == 0))
        def _():
            den_sc[...] = jnp.zeros((1, G), jnp.float32)

        @pl.when(p == 1)
        def _():
            mb = jnp.broadcast_to(m_sc[...], (BN, G))
            mpn = jnp.sum(jnp.where(eq, mb, 0.0), axis=1)
            ex = jnp.exp(sblk - mpn)
            ex_sc[j, 0, :] = ex
            den_sc[...] = den_sc[...] + jnp.sum(
                jnp.where(eq, ex[:, None], 0.0), axis=0)[None, :]

        @pl.when(p == 2)
        def _():
            db = jnp.broadcast_to(den_sc[...], (BN, G))
            dpn = jnp.sum(jnp.where(eq, db, 0.0), axis=1)
            out_ref[0, 0, :] = ex_sc[j, 0, :] / dpn

    return pl.pallas_call(
        body,
        grid=(3, NB),
        in_specs=[
            pl.BlockSpec((1, 1, BN), lambda p, j: (j, 0, 0)),
            pl.BlockSpec((1, 1, BN), lambda p, j: (j, 0, 0)),
        ],
        out_specs=pl.BlockSpec((1, 1, BN), lambda p, j: (j, 0, 0)),
        out_shape=jax.ShapeDtypeStruct((NB, 1, BN), jnp.float32),
        scratch_shapes=[
            pltpu.VMEM((1, G), jnp.float32),
            pltpu.VMEM((1, G), jnp.float32),
            pltpu.VMEM((NB, 1, BN), jnp.float32),
        ],
    )(sarr, batch3)


def kernel(x, edge_index, batch, mask, W1, b1, W2, b2, Wl1, bl1, Wl2, bl2):
    src3 = edge_index[0].reshape(NW, NCHUNK, CHUNK)
    dst3 = edge_index[1].reshape(NW, NCHUNK, CHUNK)
    zeros16 = jnp.zeros((NP, 16), jnp.float32)
    zerosD = jnp.zeros((NP, D), jnp.float32)

    degp = _deg_sc(dst3, zeros16)
    h1p = _l1_tc(x, W1, degp)
    S1 = _edge_sc(h1p, src3, dst3, zerosD)
    h2p = _l2_tc(S1, h1p, degp, b1.reshape(1, D), W2)
    S2 = _edge_sc(h2p, src3, dst3, zerosD)
    sarr = _score_tc(S2, h2p, degp, b2.reshape(1, D), Wl1,
                     bl1.reshape(1, D), Wl2, bl2.reshape(1, 1),
                     mask.reshape(NB, 1, BN))
    out3 = _softmax_tc(sarr, batch.reshape(NB, 1, BN))
    return out3.reshape(N)


# speedup vs baseline: 20.4757x; 1.0113x over previous
import functools

import jax
import jax.numpy as jnp
import numpy as np
from jax import lax
from jax.experimental import pallas as pl
from jax.experimental.pallas import tpu as pltpu
from jax.experimental.pallas import tpu_sc as plsc

N, E, D, G = 10000, 320000, 128, 64
NC, NS = 2, 16
NW = NC * NS
EPT = E // NW
CHUNK = 80
NCHUNK = EPT // CHUNK
NP = 10240
RPT = NP // NS
BN = 400
NB = N // BN
NEG = np.float32(-1000000000.0)

def _mesh():
    return plsc.VectorSubcoreMesh(core_axis_name="c", subcore_axis_name="s")


def _deg_sc(dst3, zeros1):
    @functools.partial(
        pl.kernel,
        out_type=jax.ShapeDtypeStruct((NC, NP), jnp.float32),
        mesh=_mesh(),
        scratch_types=[
            pltpu.VMEM((NCHUNK, CHUNK), jnp.int32),
            pltpu.VMEM((CHUNK,), jnp.float32),
            pltpu.VMEM_SHARED((NP,), jnp.float32),
        ],
    )
    def k(dst_hbm, z_hbm, out_hbm, idx_v, ones_v, acc_sh):
        c = lax.axis_index("c")
        s = lax.axis_index("s")
        tile = c * NS + s

        @pl.loop(0, CHUNK // 16)
        def _(i):
            ones_v[pl.ds(i * 16, 16)] = jnp.full((16,), 1.0, jnp.float32)

        @pl.when(s == 0)
        def _():
            pltpu.sync_copy(z_hbm, acc_sh)

        pltpu.sync_copy(dst_hbm.at[tile], idx_v)
        plsc.subcore_barrier()

        @pl.loop(0, NCHUNK)
        def _(j):
            pltpu.sync_copy(ones_v, acc_sh.at[idx_v.at[j]], add=True)

        plsc.subcore_barrier()

        @pl.when(s == 0)
        def _():
            pltpu.sync_copy(acc_sh, out_hbm.at[c])

    return k(dst3, zeros1)


def _edge_sc(hp, src3, dst3, zerosD):
    @functools.partial(
        pl.kernel,
        out_type=jax.ShapeDtypeStruct((NC, NP, D), jnp.float32),
        mesh=_mesh(),
        scratch_types=[
            pltpu.VMEM((CHUNK,), jnp.int32),
            pltpu.VMEM((CHUNK,), jnp.int32),
            pltpu.VMEM((CHUNK,), jnp.int32),
            pltpu.VMEM((CHUNK,), jnp.int32),
            pltpu.VMEM((CHUNK, D), jnp.float32),
            pltpu.VMEM((CHUNK, D), jnp.float32),
            pltpu.VMEM_SHARED((NP, D), jnp.float32),
            pltpu.SemaphoreType.DMA,
            pltpu.SemaphoreType.DMA,
            pltpu.SemaphoreType.DMA,
            pltpu.SemaphoreType.DMA,
            pltpu.SemaphoreType.DMA,
            pltpu.SemaphoreType.DMA,
        ],
    )
    def k(hp_hbm, src_hbm, dst_hbm, z_hbm, out_hbm,
          ibs0, ibs1, ibd0, ibd1, rows0, rows1, acc_sh,
          is0, is1, id0, id1, g0, g1):
        c = lax.axis_index("c")
        s = lax.axis_index("s")
        tile = c * NS + s
        tbase = tile * NCHUNK
        ibs = (ibs0, ibs1)
        ibd = (ibd0, ibd1)
        rows = (rows0, rows1)
        isem = (is0, is1)
        dsem = (id0, id1)
        gsem = (g0, g1)

        @pl.when(s == 0)
        def _():
            pltpu.sync_copy(z_hbm, acc_sh)

        plsc.subcore_barrier()

        def idx_start(j, b):
            pltpu.async_copy(src_hbm.at[tbase + j], ibs[b], isem[b])
            pltpu.async_copy(dst_hbm.at[tbase + j], ibd[b], dsem[b])

        def idx_wait_s(j, b):
            pltpu.make_async_copy(src_hbm.at[tbase + j], ibs[b], isem[b]).wait()

        def idx_wait_d(j, b):
            pltpu.make_async_copy(dst_hbm.at[tbase + j], ibd[b], dsem[b]).wait()

        def gather_start(b):
            pltpu.async_copy(hp_hbm.at[ibs[b]], rows[b], gsem[b])

        def gather_wait(b):
            pltpu.make_async_copy(hp_hbm.at[ibs[b]], rows[b], gsem[b]).wait()

        idx_start(0, 0)
        idx_start(1, 1)
        idx_wait_s(0, 0)
        gather_start(0)

        @pl.loop(0, NCHUNK // 2)
        def _(g):
            j = 2 * g
            gather_wait(0)
            idx_wait_s(j + 1, 1)
            gather_start(1)
            idx_wait_d(j, 0)
            pltpu.sync_copy(rows0, acc_sh.at[ibd0], add=True)
            idx_start(j + 2, 0)
            gather_wait(1)

            @pl.when(j + 2 < NCHUNK)
            def _():
                idx_wait_s(j + 2, 0)
                gather_start(0)
            idx_wait_d(j + 1, 1)
            pltpu.sync_copy(rows1, acc_sh.at[ibd1], add=True)

            @pl.when(j + 3 < NCHUNK)
            def _():
                idx_start(j + 3, 1)

        gather_wait(0)
        idx_wait_d(NCHUNK - 1, 0)
        pltpu.sync_copy(rows0, acc_sh.at[ibd0], add=True)

        plsc.subcore_barrier()

        @pl.when(s == 0)
        def _():
            pltpu.sync_copy(acc_sh, out_hbm.at[c])

    return k(hp, src3.reshape(NW * NCHUNK, CHUNK),
             dst3.reshape(NW * NCHUNK, CHUNK), zerosD)


def _l1_tc(x, W1, degp):
    def body(x_ref, w_ref, deg_ref, out_ref):
        d = deg_ref[0, :, 0] + deg_ref[1, :, 0] + 1.0
        dis = lax.rsqrt(d)
        h = jnp.dot(x_ref[...], w_ref[...],
                    preferred_element_type=jnp.float32,
                    precision=lax.Precision.HIGHEST)
        out_ref[...] = h * dis[:, None]

    return pl.pallas_call(
        body,
        grid=(NB,),
        in_specs=[
            pl.BlockSpec((BN, D), lambda i: (i, 0)),
            pl.BlockSpec((D, D), lambda i: (0, 0)),
            pl.BlockSpec((NC, BN, 1), lambda i: (0, i, 0)),
        ],
        out_specs=pl.BlockSpec((BN, D), lambda i: (i, 0)),
        out_shape=jax.ShapeDtypeStruct((N, D), jnp.float32),
    )(x, W1, degp)


def _l2_tc(S1, h1p, degp, b1r, W2):
    def body(s1_ref, h1p_ref, deg_ref, b1_ref, w_ref, out_ref):
        d = deg_ref[0, :, 0] + deg_ref[1, :, 0] + 1.0
        dis = lax.rsqrt(d)
        tot = s1_ref[0] + s1_ref[1] + h1p_ref[...]
        h1 = jnp.maximum(tot * dis[:, None] + b1_ref[...], 0.0)
        h2 = jnp.dot(h1, w_ref[...],
                     preferred_element_type=jnp.float32,
                     precision=lax.Precision.HIGHEST)
        out_ref[...] = h2 * dis[:, None]

    return pl.pallas_call(
        body,
        grid=(NB,),
        in_specs=[
            pl.BlockSpec((NC, BN, D), lambda i: (0, i, 0)),
            pl.BlockSpec((BN, D), lambda i: (i, 0)),
            pl.BlockSpec((NC, BN, 1), lambda i: (0, i, 0)),
            pl.BlockSpec((1, D), lambda i: (0, 0)),
            pl.BlockSpec((D, D), lambda i: (0, 0)),
        ],
        out_specs=pl.BlockSpec((BN, D), lambda i: (i, 0)),
        out_shape=jax.ShapeDtypeStruct((N, D), jnp.float32),
    )(S1, h1p, degp, b1r, W2)


def _head_tc(S2, h2p, degp, b2r, Wl1, bl1r, Wl2, bl2r, mask3, batch3):
    def body(s2_ref, h2p_ref, deg_ref, b2_ref, wl1_ref, bl1_ref, wl2_ref,
             bl2_ref, m_ref, b_ref, out_ref, s_sc, m_sc, den_sc, ex_sc):
        p = pl.program_id(0)
        j = pl.program_id(1)
        bblk = b_ref[0, 0, :]
        eq = bblk[:, None] == lax.broadcasted_iota(jnp.int32, (BN, G), 1)

        @pl.when(p == 0)
        def _():
            d = deg_ref[0, :, 0] + deg_ref[1, :, 0] + 1.0
            dis = lax.rsqrt(d)
            tot = s2_ref[0] + s2_ref[1] + h2p_ref[...]
            h2 = tot * dis[:, None] + b2_ref[...]
            t = jnp.maximum(
                jnp.dot(h2, wl1_ref[...],
                        preferred_element_type=jnp.float32,
                        precision=lax.Precision.HIGHEST) + bl1_ref[...], 0.0)
            sv = jnp.dot(t, wl2_ref[...],
                         preferred_element_type=jnp.float32,
                         precision=lax.Precision.HIGHEST)
            sc = sv[:, 0] + bl2_ref[0, 0]
            sc = jnp.where(m_ref[0, 0, :] == 0, NEG, sc)
            s_sc[j, 0, :] = sc

            @pl.when(j == 0)
            def _():
                m_sc[...] = jnp.full((1, G), -3.0e38, jnp.float32)

            contrib = jnp.max(jnp.where(eq, sc[:, None], -3.0e38), axis=0)
            m_sc[...] = jnp.maximum(m_sc[...], contrib[None, :])

        @pl.when(jnp.logical_and(p == 1, j == 0))
        def _():
            den_sc[...] = jnp.zeros((1, G), jnp.float32)

        @pl.when(p == 1)
        def _():
            sblk = s_sc[j, 0, :]
            mb = jnp.broadcast_to(m_sc[...], (BN, G))
            mpn = jnp.sum(jnp.where(eq, mb, 0.0), axis=1)
            ex = jnp.exp(sblk - mpn)
            ex_sc[j, 0, :] = ex
            den_sc[...] = den_sc[...] + jnp.sum(
                jnp.where(eq, ex[:, None], 0.0), axis=0)[None, :]

        @pl.when(p == 2)
        def _():
            db = jnp.broadcast_to(den_sc[...], (BN, G))
            dpn = jnp.sum(jnp.where(eq, db, 0.0), axis=1)
            out_ref[0, 0, :] = ex_sc[j, 0, :] / dpn

    return pl.pallas_call(
        body,
        grid=(3, NB),
        in_specs=[
            pl.BlockSpec((NC, BN, D), lambda p, j: (0, j, 0)),
            pl.BlockSpec((BN, D), lambda p, j: (j, 0)),
            pl.BlockSpec((NC, BN, 1), lambda p, j: (0, j, 0)),
            pl.BlockSpec((1, D), lambda p, j: (0, 0)),
            pl.BlockSpec((D, D), lambda p, j: (0, 0)),
            pl.BlockSpec((1, D), lambda p, j: (0, 0)),
            pl.BlockSpec((D, 1), lambda p, j: (0, 0)),
            pl.BlockSpec((1, 1), lambda p, j: (0, 0)),
            pl.BlockSpec((1, 1, BN), lambda p, j: (j, 0, 0)),
            pl.BlockSpec((1, 1, BN), lambda p, j: (j, 0, 0)),
        ],
        out_specs=pl.BlockSpec((1, 1, BN), lambda p, j: (j, 0, 0)),
        out_shape=jax.ShapeDtypeStruct((NB, 1, BN), jnp.float32),
        scratch_shapes=[
            pltpu.VMEM((NB, 1, BN), jnp.float32),
            pltpu.VMEM((1, G), jnp.float32),
            pltpu.VMEM((1, G), jnp.float32),
            pltpu.VMEM((NB, 1, BN), jnp.float32),
        ],
    )(S2, h2p, degp, b2r, Wl1, bl1r, Wl2, bl2r, mask3, batch3)


def kernel(x, edge_index, batch, mask, W1, b1, W2, b2, Wl1, bl1, Wl2, bl2):
    src3 = edge_index[0].reshape(NW, NCHUNK, CHUNK)
    dst3 = edge_index[1].reshape(NW, NCHUNK, CHUNK)
    zeros1 = jnp.zeros((NP,), jnp.float32)
    zerosD = jnp.zeros((NP, D), jnp.float32)

    degp = _deg_sc(dst3, zeros1)
    degp3 = degp.reshape(NC, NP, 1)
    h1p = _l1_tc(x, W1, degp3)
    S1 = _edge_sc(h1p, src3, dst3, zerosD)
    h2p = _l2_tc(S1, h1p, degp3, b1.reshape(1, D), W2)
    S2 = _edge_sc(h2p, src3, dst3, zerosD)
    out3 = _head_tc(S2, h2p, degp3, b2.reshape(1, D), Wl1,
                    bl1.reshape(1, D), Wl2, bl2.reshape(1, 1),
                    mask.reshape(NB, 1, BN), batch.reshape(NB, 1, BN))
    return out3.reshape(N)
